# baseline (speedup 1.0000x reference)
import dataclasses
import functools

import jax
import jax.numpy as jnp
from jax import lax
from jax.experimental import pallas as pl
from jax.experimental.pallas import tpu as pltpu
from jax.experimental.pallas import tpu_sc as plsc

N = 10000
E = 320000
D = 128
H = 8
C = 16
ED = 16

BN = 1000
BE = 4000
CH = 4000
NW = 32
QC = 4

def _mesh():
    return plsc.VectorSubcoreMesh(
        core_axis_name="c", subcore_axis_name="s",
        num_cores=2, num_subcores=16)


def _sc_params():
    cp = pltpu.CompilerParams()
    if "needs_layout_passes" in pltpu.CompilerParams.__dataclass_fields__:
        cp = dataclasses.replace(cp, needs_layout_passes=False)
    return cp


def _zero_ref(ref, n):
    @pl.loop(0, n, step=16)
    def _(i):
        ref[pl.ds(i, 16)] = jnp.zeros((16,), jnp.float32)


def _ds8(off, size):
    return pl.ds(pl.multiple_of(off, 8), size)


def _sc1_body(src_hbm, dst_hbm, eaT_hbm, aeT_hbm, asdT_hbm, mh_hbm,
              laP_hbm, degP_hbm, wT_hbm, denP_hbm,
              sb0, sb1, db0, db1, cb0, cb1, ab0, ab1, wb0, wb1,
              la_acc, deg_acc, asrc_t, adst_t, den_acc, mh_v,
              sem0, sem1, wsem0, wsem1):
    wid = lax.axis_index("s") * 2 + lax.axis_index("c")
    sbs = (sb0, sb1)
    dbs = (db0, db1)
    cbs = (cb0, cb1)
    abs_ = (ab0, ab1)
    wbs = (wb0, wb1)
    sems = (sem0, sem1)
    wsems = (wsem0, wsem1)

    ec = wid % 16
    pa = wid // 16
    base_a = pa * (E // 2)
    n_a = (E // 2) // CH

    def issue_a(ci, b):
        off = base_a + ci * CH
        pltpu.async_copy(dst_hbm.at[_ds8(off, CH)], dbs[b], sems[b])
        pltpu.async_copy(eaT_hbm.at[_ds8(ec * E + off, CH)], cbs[b], sems[b])

    def drain_a(b):
        pltpu.make_async_copy(dst_hbm.at[_ds8(0, CH)], dbs[b], sems[b]).wait()
        pltpu.make_async_copy(eaT_hbm.at[_ds8(0, CH)], cbs[b], sems[b]).wait()

    issue_a(0, 0)
    issue_a(1, 1)
    _zero_ref(la_acc, N)
    _zero_ref(deg_acc, N)

    @pl.loop(0, n_a, step=2)
    def _(ci):
        for b in range(2):
            cur = ci + b
            drain_a(b)

            @pl.loop(0, CH, step=16)
            def _(j):
                dv = dbs[b][pl.ds(j, 16)]
                plsc.addupdate_scatter(la_acc, [dv], cbs[b][pl.ds(j, 16)])

            @pl.when(ec == 0)
            def _():
                @pl.loop(0, CH, step=16)
                def _(j):
                    dv = dbs[b][pl.ds(j, 16)]
                    plsc.addupdate_scatter(deg_acc, [dv],
                                           jnp.ones((16,), jnp.float32))

            @pl.when(cur + 2 < n_a)
            def _():
                issue_a(cur + 2, b)

    pltpu.sync_copy(la_acc, laP_hbm.at[_ds8(wid * N, N)])

    @pl.when(ec == 0)
    def _():
        pltpu.sync_copy(deg_acc, degP_hbm.at[_ds8(pa * N, N)])

    h = wid // 4
    pb = wid % 4
    base_b = pb * (E // 4)
    n_b = (E // 4) // CH

    def issue_b(ci, b):
        off = base_b + ci * CH
        pltpu.async_copy(src_hbm.at[_ds8(off, CH)], sbs[b], sems[b])
        pltpu.async_copy(dst_hbm.at[_ds8(off, CH)], dbs[b], sems[b])
        pltpu.async_copy(aeT_hbm.at[_ds8(h * E + off, CH)], abs_[b], sems[b])

    def drain_b(b):
        pltpu.make_async_copy(src_hbm.at[_ds8(0, CH)], sbs[b], sems[b]).wait()
        pltpu.make_async_copy(src_hbm.at[_ds8(0, CH)], dbs[b], sems[b]).wait()
        pltpu.make_async_copy(aeT_hbm.at[_ds8(0, CH)], abs_[b], sems[b]).wait()

    def drain_w(b):
        pltpu.make_async_copy(wbs[b], wT_hbm.at[_ds8(0, CH)],
                              wsems[b]).wait()

    issue_b(0, 0)
    issue_b(1, 1)
    pltpu.sync_copy(asdT_hbm.at[_ds8(h * N, N)], asrc_t)
    pltpu.sync_copy(asdT_hbm.at[_ds8((H + h) * N, N)], adst_t)
    pltpu.sync_copy(mh_hbm.at[_ds8(h * 16, 16)], mh_v)
    _zero_ref(den_acc, N)

    @pl.loop(0, n_b, step=2)
    def _(ci):
        for b in range(2):
            cur = ci + b
            drain_b(b)

            @pl.when(cur >= 2)
            def _():
                drain_w(b)

            @pl.loop(0, CH, step=16)
            def _(j):
                sv = sbs[b][pl.ds(j, 16)]
                dv = dbs[b][pl.ds(j, 16)]
                a = (plsc.load_gather(asrc_t, [sv])
                     + plsc.load_gather(adst_t, [dv])
                     + abs_[b][pl.ds(j, 16)])
                a = jnp.maximum(a, a * 0.2)
                wv = jnp.exp(a - mh_v[...])
                wbs[b][pl.ds(j, 16)] = wv
                plsc.addupdate_scatter(den_acc, [dv], wv)

            pltpu.async_copy(wbs[b],
                             wT_hbm.at[_ds8(h * E + base_b + cur * CH, CH)],
                             wsems[b])

            @pl.when(cur + 2 < n_b)
            def _():
                issue_b(cur + 2, b)

    drain_w(0)
    drain_w(1)
    pltpu.sync_copy(den_acc, denP_hbm.at[_ds8(wid * N, N)])


def _sc1(src, dst, eaT, aeT, asdT, mh_tile):
    f = functools.partial(
        pl.kernel,
        out_type=[
            jax.ShapeDtypeStruct((NW * N,), jnp.float32),
            jax.ShapeDtypeStruct((2 * N,), jnp.float32),
            jax.ShapeDtypeStruct((H * E,), jnp.float32),
            jax.ShapeDtypeStruct((NW * N,), jnp.float32),
        ],
        mesh=_mesh(),
        compiler_params=_sc_params(),
        scratch_types=[
            pltpu.VMEM((CH,), jnp.int32),
            pltpu.VMEM((CH,), jnp.int32),
            pltpu.VMEM((CH,), jnp.int32),
            pltpu.VMEM((CH,), jnp.int32),
            pltpu.VMEM((CH,), jnp.float32),
            pltpu.VMEM((CH,), jnp.float32),
            pltpu.VMEM((CH,), jnp.float32),
            pltpu.VMEM((CH,), jnp.float32),
            pltpu.VMEM((CH,), jnp.float32),
            pltpu.VMEM((CH,), jnp.float32),
            pltpu.VMEM((N,), jnp.float32),
            pltpu.VMEM((N,), jnp.float32),
            pltpu.VMEM((N,), jnp.float32),
            pltpu.VMEM((N,), jnp.float32),
            pltpu.VMEM((N,), jnp.float32),
            pltpu.VMEM((16,), jnp.float32),
        ] + [pltpu.SemaphoreType.DMA] * 4,
    )
    return f(_sc1_body)(src, dst, eaT, aeT, asdT, mh_tile)


def _sc2_body(src_hbm, dst_hbm, wT_hbm, xlQ_hbm, outQ_hbm,
              sb0, sb1, db0, db1, wb0, wb1,
              xl0, xl1, xl2, xl3, ac0, ac1, ac2, ac3,
              sem0, sem1):
    wid = lax.axis_index("s") * 2 + lax.axis_index("c")
    h = wid // 4
    xls = (xl0, xl1, xl2, xl3)
    acs = (ac0, ac1, ac2, ac3)
    sbs = (sb0, sb1)
    dbs = (db0, db1)
    wbs = (wb0, wb1)
    sems = (sem0, sem1)
    n_out = E // CH

    def issue(ci, b):
        off = ci * CH
        pltpu.async_copy(src_hbm.at[_ds8(off, CH)], sbs[b], sems[b])
        pltpu.async_copy(dst_hbm.at[_ds8(off, CH)], dbs[b], sems[b])
        pltpu.async_copy(wT_hbm.at[_ds8(h * E + off, CH)], wbs[b], sems[b])

    def drain(b):
        pltpu.make_async_copy(src_hbm.at[_ds8(0, CH)], sbs[b], sems[b]).wait()
        pltpu.make_async_copy(src_hbm.at[_ds8(0, CH)], dbs[b], sems[b]).wait()
        pltpu.make_async_copy(wT_hbm.at[_ds8(0, CH)], wbs[b], sems[b]).wait()

    issue(0, 0)
    issue(1, 1)

    for c in range(QC):
        pltpu.sync_copy(xlQ_hbm.at[_ds8((wid * QC + c) * N, N)], xls[c])
        _zero_ref(acs[c], N)

    @pl.loop(0, n_out, step=2)
    def _(ci):
        for b in range(2):
            cur = ci + b
            drain(b)

            @pl.loop(0, CH, step=16, unroll=4)
            def _(j):
                sv = sbs[b][pl.ds(j, 16)]
                dv = dbs[b][pl.ds(j, 16)]
                wv = wbs[b][pl.ds(j, 16)]
                for c in range(QC):
                    g = plsc.load_gather(xls[c], [sv])
                    plsc.addupdate_scatter(acs[c], [dv], g * wv)

            @pl.when(cur + 2 < n_out)
            def _():
                issue(cur + 2, b)

    for c in range(QC):
        pltpu.sync_copy(acs[c], outQ_hbm.at[_ds8((wid * QC + c) * N, N)])


def _sc2(src, dst, wT, xlQ):
    f = functools.partial(
        pl.kernel,
        out_type=jax.ShapeDtypeStruct((NW * QC * N,), jnp.float32),
        mesh=_mesh(),
        compiler_params=_sc_params(),
        scratch_types=[
            pltpu.VMEM((CH,), jnp.int32),
            pltpu.VMEM((CH,), jnp.int32),
            pltpu.VMEM((CH,), jnp.int32),
            pltpu.VMEM((CH,), jnp.int32),
            pltpu.VMEM((CH,), jnp.float32),
            pltpu.VMEM((CH,), jnp.float32),
        ] + [pltpu.VMEM((N,), jnp.float32)] * 8
          + [pltpu.SemaphoreType.DMA] * 2,
    )
    return f(_sc2_body)(src, dst, wT, xlQ)


def _proj_body(x_ref, w_ref, asd_ref, xl_ref, a_ref):
    xb = x_ref[...]
    xl_ref[...] = jnp.dot(xb, w_ref[...], preferred_element_type=jnp.float32)
    a_ref[...] = jnp.dot(xb, asd_ref[...], preferred_element_type=jnp.float32)


def _ae_body(ea_ref, aemat_ref, ae_ref):
    ae_ref[...] = jnp.dot(ea_ref[...], aemat_ref[...],
                          preferred_element_type=jnp.float32)


def _post_body(m_ref, rd_ref, sl_ref, xl_ref, x_ref, b_ref, g_ref, be_ref,
               o_ref):
    out = m_ref[...] * rd_ref[...] + sl_ref[...] * xl_ref[...] + b_ref[...]
    mu = out.mean(-1, keepdims=True)
    var = ((out - mu) ** 2).mean(-1, keepdims=True)
    normed = (out - mu) / jnp.sqrt(var + 1e-5) * g_ref[...] + be_ref[...]
    z = normed + x_ref[...]
    o_ref[...] = jnp.where(z > 0, z, jnp.exp(jnp.minimum(z, 0.0)) - 1.0)


def kernel(x, edge_index, edge_attr, W, W_edge, att_src, att_dst, att_edge,
           bias, gamma, beta):
    n = x.shape[0]
    src = edge_index[0]
    dst = edge_index[1]

    A_s = (W.reshape(D, H, C) * att_src).sum(-1)
    A_d = (W.reshape(D, H, C) * att_dst).sum(-1)
    A_e = (W_edge.reshape(ED, H, C) * att_edge).sum(-1)
    ASD = jnp.concatenate([A_s, A_d], axis=1)

    xl, asd = pl.pallas_call(
        _proj_body,
        grid=(n // BN,),
        in_specs=[
            pl.BlockSpec((BN, D), lambda i: (i, 0)),
            pl.BlockSpec((D, H * C), lambda i: (0, 0)),
            pl.BlockSpec((D, 2 * H), lambda i: (0, 0)),
        ],
        out_specs=[
            pl.BlockSpec((BN, D), lambda i: (i, 0)),
            pl.BlockSpec((BN, 2 * H), lambda i: (i, 0)),
        ],
        out_shape=[
            jax.ShapeDtypeStruct((n, D), jnp.float32),
            jax.ShapeDtypeStruct((n, 2 * H), jnp.float32),
        ],
    )(x, W, ASD)

    ae = pl.pallas_call(
        _ae_body,
        grid=(E // BE,),
        in_specs=[
            pl.BlockSpec((BE, ED), lambda i: (i, 0)),
            pl.BlockSpec((ED, H), lambda i: (0, 0)),
        ],
        out_specs=pl.BlockSpec((BE, H), lambda i: (i, 0)),
        out_shape=jax.ShapeDtypeStruct((E, H), jnp.float32),
    )(edge_attr, A_e)

    asdT = asd.T.reshape(-1)
    aeT = ae.T.reshape(-1)
    eaT = edge_attr.T.reshape(-1)
    xlQ = xl.reshape(n, NW, QC).transpose(1, 2, 0).reshape(-1)

    mh_raw = (jnp.max(asd[:, :H], 0) + jnp.max(asd[:, H:], 0)
              + jnp.max(ae, 0))
    mh = jax.nn.leaky_relu(mh_raw, negative_slope=0.2)
    mh_tile = jnp.tile(mh[:, None], (1, 16)).reshape(-1)

    laP, degP, wT, denP = _sc1(src, dst, eaT, aeT, asdT, mh_tile)
    laP = laP.reshape(NW, n)
    degP = degP.reshape(2, n)
    denP = denP.reshape(NW, n)

    deg = degP.sum(0)
    la = (laP[:16] + laP[16:]) / jnp.clip(deg, 1.0)
    ael = la.T @ A_e
    a_loop = asd[:, :H] + asd[:, H:] + ael
    a_loop = jax.nn.leaky_relu(a_loop, negative_slope=0.2)
    w_loop = jnp.exp(a_loop - mh[None, :])
    denT = denP.reshape(H, 4, n).sum(1) + w_loop.T
    rd2 = 1.0 / denT

    outQ = _sc2(src, dst, wT, xlQ)
    msgP = outQ.reshape(NW, QC, n).transpose(2, 0, 1).reshape(n, D)

    sl = w_loop * rd2.T
    slexp = jnp.repeat(sl, C, axis=1)
    rdexp = jnp.repeat(rd2.T, C, axis=1)

    out = pl.pallas_call(
        _post_body,
        grid=(n // BN,),
        in_specs=[
            pl.BlockSpec((BN, D), lambda i: (i, 0)),
            pl.BlockSpec((BN, D), lambda i: (i, 0)),
            pl.BlockSpec((BN, D), lambda i: (i, 0)),
            pl.BlockSpec((BN, D), lambda i: (i, 0)),
            pl.BlockSpec((BN, D), lambda i: (i, 0)),
            pl.BlockSpec((1, D), lambda i: (0, 0)),
            pl.BlockSpec((1, D), lambda i: (0, 0)),
            pl.BlockSpec((1, D), lambda i: (0, 0)),
        ],
        out_specs=pl.BlockSpec((BN, D), lambda i: (i, 0)),
        out_shape=jax.ShapeDtypeStruct((n, D), jnp.float32),
    )(msgP, rdexp, slexp, xl, x, bias.reshape(1, D), gamma.reshape(1, D),
      beta.reshape(1, D))
    return out


# speedup vs baseline: 1.4394x; 1.4394x over previous
import dataclasses
import functools

import jax
import jax.numpy as jnp
from jax import lax
from jax.experimental import pallas as pl
from jax.experimental.pallas import tpu as pltpu
from jax.experimental.pallas import tpu_sc as plsc

N = 10000
E = 320000
D = 128
H = 8
C = 16
ED = 16

BN = 1000
BE = 4000
CH = 4000
NW = 32
QC = 4

def _mesh():
    return plsc.VectorSubcoreMesh(
        core_axis_name="c", subcore_axis_name="s",
        num_cores=2, num_subcores=16)


def _sc_params():
    cp = pltpu.CompilerParams()
    if "needs_layout_passes" in pltpu.CompilerParams.__dataclass_fields__:
        cp = dataclasses.replace(cp, needs_layout_passes=False)
    return cp


def _zero_ref(ref, n):
    @pl.loop(0, n, step=16)
    def _(i):
        ref[pl.ds(i, 16)] = jnp.zeros((16,), jnp.float32)


def _ds8(off, size):
    return pl.ds(pl.multiple_of(off, 8), size)


def _sc1_body(src_hbm, dst_hbm, eaT_hbm, aeT_hbm, asdT_hbm, mh_hbm,
              laP_hbm, degP_hbm, wT_hbm, denP_hbm,
              sb0, sb1, db0, db1, cb0, cb1, ab0, ab1, wb0, wb1,
              la_acc, deg_acc, asrc_t, adst_t, den_acc, mh_v,
              sem0, sem1, wsem0, wsem1):
    wid = lax.axis_index("s") * 2 + lax.axis_index("c")
    sbs = (sb0, sb1)
    dbs = (db0, db1)
    cbs = (cb0, cb1)
    abs_ = (ab0, ab1)
    wbs = (wb0, wb1)
    sems = (sem0, sem1)
    wsems = (wsem0, wsem1)

    ec = wid % 16
    pa = wid // 16
    base_a = pa * (E // 2)
    n_a = (E // 2) // CH

    def issue_a(ci, b):
        off = base_a + ci * CH
        pltpu.async_copy(dst_hbm.at[_ds8(off, CH)], dbs[b], sems[b])
        pltpu.async_copy(eaT_hbm.at[_ds8(ec * E + off, CH)], cbs[b], sems[b])

    def drain_a(b):
        pltpu.make_async_copy(dst_hbm.at[_ds8(0, CH)], dbs[b], sems[b]).wait()
        pltpu.make_async_copy(eaT_hbm.at[_ds8(0, CH)], cbs[b], sems[b]).wait()

    issue_a(0, 0)
    issue_a(1, 1)
    _zero_ref(la_acc, N)
    _zero_ref(deg_acc, N)

    @pl.loop(0, n_a, step=2)
    def _(ci):
        for b in range(2):
            cur = ci + b
            drain_a(b)

            @pl.loop(0, CH, step=16)
            def _(j):
                dv = dbs[b][pl.ds(j, 16)]
                plsc.addupdate_scatter(la_acc, [dv], cbs[b][pl.ds(j, 16)])

            @pl.when(ec == 0)
            def _():
                @pl.loop(0, CH, step=16)
                def _(j):
                    dv = dbs[b][pl.ds(j, 16)]
                    plsc.addupdate_scatter(deg_acc, [dv],
                                           jnp.ones((16,), jnp.float32))

            @pl.when(cur + 2 < n_a)
            def _():
                issue_a(cur + 2, b)

    pltpu.sync_copy(la_acc, laP_hbm.at[_ds8(wid * N, N)])

    @pl.when(ec == 0)
    def _():
        pltpu.sync_copy(deg_acc, degP_hbm.at[_ds8(pa * N, N)])

    h = wid // 4
    pb = wid % 4
    base_b = pb * (E // 4)
    n_b = (E // 4) // CH

    def issue_b(ci, b):
        off = base_b + ci * CH
        pltpu.async_copy(src_hbm.at[_ds8(off, CH)], sbs[b], sems[b])
        pltpu.async_copy(dst_hbm.at[_ds8(off, CH)], dbs[b], sems[b])
        pltpu.async_copy(aeT_hbm.at[_ds8(h * E + off, CH)], abs_[b], sems[b])

    def drain_b(b):
        pltpu.make_async_copy(src_hbm.at[_ds8(0, CH)], sbs[b], sems[b]).wait()
        pltpu.make_async_copy(src_hbm.at[_ds8(0, CH)], dbs[b], sems[b]).wait()
        pltpu.make_async_copy(aeT_hbm.at[_ds8(0, CH)], abs_[b], sems[b]).wait()

    def drain_w(b):
        pltpu.make_async_copy(wbs[b], wT_hbm.at[_ds8(0, CH)],
                              wsems[b]).wait()

    issue_b(0, 0)
    issue_b(1, 1)
    pltpu.sync_copy(asdT_hbm.at[_ds8(h * N, N)], asrc_t)
    pltpu.sync_copy(asdT_hbm.at[_ds8((H + h) * N, N)], adst_t)
    pltpu.sync_copy(mh_hbm.at[_ds8(h * 16, 16)], mh_v)
    _zero_ref(den_acc, N)

    @pl.loop(0, n_b, step=2)
    def _(ci):
        for b in range(2):
            cur = ci + b
            drain_b(b)

            @pl.when(cur >= 2)
            def _():
                drain_w(b)

            @pl.loop(0, CH, step=16)
            def _(j):
                sv = sbs[b][pl.ds(j, 16)]
                dv = dbs[b][pl.ds(j, 16)]
                a = (plsc.load_gather(asrc_t, [sv])
                     + plsc.load_gather(adst_t, [dv])
                     + abs_[b][pl.ds(j, 16)])
                a = jnp.maximum(a, a * 0.2)
                wv = jnp.exp(a - mh_v[...])
                wbs[b][pl.ds(j, 16)] = wv
                plsc.addupdate_scatter(den_acc, [dv], wv)

            pltpu.async_copy(wbs[b],
                             wT_hbm.at[_ds8(h * E + base_b + cur * CH, CH)],
                             wsems[b])

            @pl.when(cur + 2 < n_b)
            def _():
                issue_b(cur + 2, b)

    drain_w(0)
    drain_w(1)
    pltpu.sync_copy(den_acc, denP_hbm.at[_ds8(wid * N, N)])


def _sc1(src, dst, eaT, aeT, asdT, mh_tile):
    f = functools.partial(
        pl.kernel,
        out_type=[
            jax.ShapeDtypeStruct((NW * N,), jnp.float32),
            jax.ShapeDtypeStruct((2 * N,), jnp.float32),
            jax.ShapeDtypeStruct((H * E,), jnp.float32),
            jax.ShapeDtypeStruct((NW * N,), jnp.float32),
        ],
        mesh=_mesh(),
        compiler_params=_sc_params(),
        scratch_types=[
            pltpu.VMEM((CH,), jnp.int32),
            pltpu.VMEM((CH,), jnp.int32),
            pltpu.VMEM((CH,), jnp.int32),
            pltpu.VMEM((CH,), jnp.int32),
            pltpu.VMEM((CH,), jnp.float32),
            pltpu.VMEM((CH,), jnp.float32),
            pltpu.VMEM((CH,), jnp.float32),
            pltpu.VMEM((CH,), jnp.float32),
            pltpu.VMEM((CH,), jnp.float32),
            pltpu.VMEM((CH,), jnp.float32),
            pltpu.VMEM((N,), jnp.float32),
            pltpu.VMEM((N,), jnp.float32),
            pltpu.VMEM((N,), jnp.float32),
            pltpu.VMEM((N,), jnp.float32),
            pltpu.VMEM((N,), jnp.float32),
            pltpu.VMEM((16,), jnp.float32),
        ] + [pltpu.SemaphoreType.DMA] * 4,
    )
    return f(_sc1_body)(src, dst, eaT, aeT, asdT, mh_tile)


def _sc2_body(src_hbm, dst_hbm, wT_hbm, xlQ_hbm, outQ_hbm,
              sb0, sb1, db0, db1, wb0, wb1,
              xl0, xl1, xl2, xl3, ac0, ac1, ac2, ac3,
              sem0, sem1):
    wid = lax.axis_index("s") * 2 + lax.axis_index("c")
    h = wid // 4
    xls = (xl0, xl1, xl2, xl3)
    acs = (ac0, ac1, ac2, ac3)
    sbs = (sb0, sb1)
    dbs = (db0, db1)
    wbs = (wb0, wb1)
    sems = (sem0, sem1)
    n_out = E // CH

    def issue(ci, b):
        off = ci * CH
        pltpu.async_copy(src_hbm.at[_ds8(off, CH)], sbs[b], sems[b])
        pltpu.async_copy(dst_hbm.at[_ds8(off, CH)], dbs[b], sems[b])
        pltpu.async_copy(wT_hbm.at[_ds8(h * E + off, CH)], wbs[b], sems[b])

    def drain(b):
        pltpu.make_async_copy(src_hbm.at[_ds8(0, CH)], sbs[b], sems[b]).wait()
        pltpu.make_async_copy(src_hbm.at[_ds8(0, CH)], dbs[b], sems[b]).wait()
        pltpu.make_async_copy(wT_hbm.at[_ds8(0, CH)], wbs[b], sems[b]).wait()

    issue(0, 0)
    issue(1, 1)

    for c in range(QC):
        pltpu.sync_copy(xlQ_hbm.at[_ds8((wid * QC + c) * N, N)], xls[c])
        _zero_ref(acs[c], N)

    @pl.loop(0, n_out, step=2)
    def _(ci):
        for b in range(2):
            cur = ci + b
            drain(b)

            @plsc.parallel_loop(0, CH, step=16, unroll=4)
            def _(j):
                sv = sbs[b][pl.ds(j, 16)]
                dv = dbs[b][pl.ds(j, 16)]
                wv = wbs[b][pl.ds(j, 16)]
                for c in range(QC):
                    g = plsc.load_gather(xls[c], [sv])
                    plsc.addupdate_scatter(acs[c], [dv], g * wv)

            @pl.when(cur + 2 < n_out)
            def _():
                issue(cur + 2, b)

    for c in range(QC):
        pltpu.sync_copy(acs[c], outQ_hbm.at[_ds8((wid * QC + c) * N, N)])


def _sc2(src, dst, wT, xlQ):
    f = functools.partial(
        pl.kernel,
        out_type=jax.ShapeDtypeStruct((NW * QC * N,), jnp.float32),
        mesh=_mesh(),
        compiler_params=_sc_params(),
        scratch_types=[
            pltpu.VMEM((CH,), jnp.int32),
            pltpu.VMEM((CH,), jnp.int32),
            pltpu.VMEM((CH,), jnp.int32),
            pltpu.VMEM((CH,), jnp.int32),
            pltpu.VMEM((CH,), jnp.float32),
            pltpu.VMEM((CH,), jnp.float32),
        ] + [pltpu.VMEM((N,), jnp.float32)] * 8
          + [pltpu.SemaphoreType.DMA] * 2,
    )
    return f(_sc2_body)(src, dst, wT, xlQ)


def _proj_body(x_ref, w_ref, asd_ref, xl_ref, a_ref):
    xb = x_ref[...]
    xl_ref[...] = jnp.dot(xb, w_ref[...], preferred_element_type=jnp.float32)
    a_ref[...] = jnp.dot(xb, asd_ref[...], preferred_element_type=jnp.float32)


def _ae_body(ea_ref, aemat_ref, ae_ref):
    ae_ref[...] = jnp.dot(ea_ref[...], aemat_ref[...],
                          preferred_element_type=jnp.float32)


def _post_body(m_ref, rd_ref, sl_ref, xl_ref, x_ref, b_ref, g_ref, be_ref,
               o_ref):
    out = m_ref[...] * rd_ref[...] + sl_ref[...] * xl_ref[...] + b_ref[...]
    mu = out.mean(-1, keepdims=True)
    var = ((out - mu) ** 2).mean(-1, keepdims=True)
    normed = (out - mu) / jnp.sqrt(var + 1e-5) * g_ref[...] + be_ref[...]
    z = normed + x_ref[...]
    o_ref[...] = jnp.where(z > 0, z, jnp.exp(jnp.minimum(z, 0.0)) - 1.0)


def kernel(x, edge_index, edge_attr, W, W_edge, att_src, att_dst, att_edge,
           bias, gamma, beta):
    n = x.shape[0]
    src = edge_index[0]
    dst = edge_index[1]

    A_s = (W.reshape(D, H, C) * att_src).sum(-1)
    A_d = (W.reshape(D, H, C) * att_dst).sum(-1)
    A_e = (W_edge.reshape(ED, H, C) * att_edge).sum(-1)
    ASD = jnp.concatenate([A_s, A_d], axis=1)

    xl, asd = pl.pallas_call(
        _proj_body,
        grid=(n // BN,),
        in_specs=[
            pl.BlockSpec((BN, D), lambda i: (i, 0)),
            pl.BlockSpec((D, H * C), lambda i: (0, 0)),
            pl.BlockSpec((D, 2 * H), lambda i: (0, 0)),
        ],
        out_specs=[
            pl.BlockSpec((BN, D), lambda i: (i, 0)),
            pl.BlockSpec((BN, 2 * H), lambda i: (i, 0)),
        ],
        out_shape=[
            jax.ShapeDtypeStruct((n, D), jnp.float32),
            jax.ShapeDtypeStruct((n, 2 * H), jnp.float32),
        ],
    )(x, W, ASD)

    ae = pl.pallas_call(
        _ae_body,
        grid=(E // BE,),
        in_specs=[
            pl.BlockSpec((BE, ED), lambda i: (i, 0)),
            pl.BlockSpec((ED, H), lambda i: (0, 0)),
        ],
        out_specs=pl.BlockSpec((BE, H), lambda i: (i, 0)),
        out_shape=jax.ShapeDtypeStruct((E, H), jnp.float32),
    )(edge_attr, A_e)

    asdT = asd.T.reshape(-1)
    aeT = ae.T.reshape(-1)
    eaT = edge_attr.T.reshape(-1)
    xlQ = xl.reshape(n, NW, QC).transpose(1, 2, 0).reshape(-1)

    mh_raw = (jnp.max(asd[:, :H], 0) + jnp.max(asd[:, H:], 0)
              + jnp.max(ae, 0))
    mh = jax.nn.leaky_relu(mh_raw, negative_slope=0.2)
    mh_tile = jnp.tile(mh[:, None], (1, 16)).reshape(-1)

    laP, degP, wT, denP = _sc1(src, dst, eaT, aeT, asdT, mh_tile)
    laP = laP.reshape(NW, n)
    degP = degP.reshape(2, n)
    denP = denP.reshape(NW, n)

    deg = degP.sum(0)
    la = (laP[:16] + laP[16:]) / jnp.clip(deg, 1.0)
    ael = la.T @ A_e
    a_loop = asd[:, :H] + asd[:, H:] + ael
    a_loop = jax.nn.leaky_relu(a_loop, negative_slope=0.2)
    w_loop = jnp.exp(a_loop - mh[None, :])
    denT = denP.reshape(H, 4, n).sum(1) + w_loop.T
    rd2 = 1.0 / denT

    outQ = _sc2(src, dst, wT, xlQ)
    msgP = outQ.reshape(NW, QC, n).transpose(2, 0, 1).reshape(n, D)

    sl = w_loop * rd2.T
    slexp = jnp.repeat(sl, C, axis=1)
    rdexp = jnp.repeat(rd2.T, C, axis=1)

    out = pl.pallas_call(
        _post_body,
        grid=(n // BN,),
        in_specs=[
            pl.BlockSpec((BN, D), lambda i: (i, 0)),
            pl.BlockSpec((BN, D), lambda i: (i, 0)),
            pl.BlockSpec((BN, D), lambda i: (i, 0)),
            pl.BlockSpec((BN, D), lambda i: (i, 0)),
            pl.BlockSpec((BN, D), lambda i: (i, 0)),
            pl.BlockSpec((1, D), lambda i: (0, 0)),
            pl.BlockSpec((1, D), lambda i: (0, 0)),
            pl.BlockSpec((1, D), lambda i: (0, 0)),
        ],
        out_specs=pl.BlockSpec((BN, D), lambda i: (i, 0)),
        out_shape=jax.ShapeDtypeStruct((n, D), jnp.float32),
    )(msgP, rdexp, slexp, xl, x, bias.reshape(1, D), gamma.reshape(1, D),
      beta.reshape(1, D))
    return out


# speedup vs baseline: 1.7794x; 1.2362x over previous
import dataclasses
import functools

import jax
import jax.numpy as jnp
from jax import lax
from jax.experimental import pallas as pl
from jax.experimental.pallas import tpu as pltpu
from jax.experimental.pallas import tpu_sc as plsc

N = 10000
E = 320000
D = 128
H = 8
C = 16
ED = 16

BN = 1000
BE = 4000
CH = 4000
NW = 32
QC = 4

def _mesh():
    return plsc.VectorSubcoreMesh(
        core_axis_name="c", subcore_axis_name="s",
        num_cores=2, num_subcores=16)


def _sc_params():
    cp = pltpu.CompilerParams()
    if "needs_layout_passes" in pltpu.CompilerParams.__dataclass_fields__:
        cp = dataclasses.replace(cp, needs_layout_passes=False)
    return cp


def _zero_ref(ref, n):
    @pl.loop(0, n, step=16)
    def _(i):
        ref[pl.ds(i, 16)] = jnp.zeros((16,), jnp.float32)


def _ds8(off, size):
    return pl.ds(pl.multiple_of(off, 8), size)


def _sc1_body(src_hbm, dst_hbm, eaT_hbm, aeT_hbm, asdT_hbm, mh_hbm,
              laP_hbm, degP_hbm, wT_hbm, denP_hbm,
              sb0, sb1, db0, db1, cb0, cb1, ab0, ab1, wb0, wb1,
              la_acc, deg_acc, asrc_t, adst_t, den_acc, mh_v,
              sem0, sem1, wsem0, wsem1):
    wid = lax.axis_index("s") * 2 + lax.axis_index("c")
    sbs = (sb0, sb1)
    dbs = (db0, db1)
    cbs = (cb0, cb1)
    abs_ = (ab0, ab1)
    wbs = (wb0, wb1)
    sems = (sem0, sem1)
    wsems = (wsem0, wsem1)

    ec = wid % 16
    pa = wid // 16
    base_a = pa * (E // 2)
    n_a = (E // 2) // CH

    def issue_a(ci, b):
        off = base_a + ci * CH
        pltpu.async_copy(dst_hbm.at[_ds8(off, CH)], dbs[b], sems[b])
        pltpu.async_copy(eaT_hbm.at[_ds8(ec * E + off, CH)], cbs[b], sems[b])

    def drain_a(b):
        pltpu.make_async_copy(dst_hbm.at[_ds8(0, CH)], dbs[b], sems[b]).wait()
        pltpu.make_async_copy(eaT_hbm.at[_ds8(0, CH)], cbs[b], sems[b]).wait()

    issue_a(0, 0)
    issue_a(1, 1)
    _zero_ref(la_acc, N)
    _zero_ref(deg_acc, N)

    @pl.loop(0, n_a, step=2)
    def _(ci):
        for b in range(2):
            cur = ci + b
            drain_a(b)

            @plsc.parallel_loop(0, CH, step=16, unroll=4)
            def _(j):
                dv = dbs[b][pl.ds(j, 16)]
                plsc.addupdate_scatter(la_acc, [dv], cbs[b][pl.ds(j, 16)])

            @pl.when(ec == 0)
            def _():
                @plsc.parallel_loop(0, CH, step=16, unroll=4)
                def _(j):
                    dv = dbs[b][pl.ds(j, 16)]
                    plsc.addupdate_scatter(deg_acc, [dv],
                                           jnp.ones((16,), jnp.float32))

            @pl.when(cur + 2 < n_a)
            def _():
                issue_a(cur + 2, b)

    pltpu.sync_copy(la_acc, laP_hbm.at[_ds8(wid * N, N)])

    @pl.when(ec == 0)
    def _():
        pltpu.sync_copy(deg_acc, degP_hbm.at[_ds8(pa * N, N)])

    h = wid // 4
    pb = wid % 4
    base_b = pb * (E // 4)
    n_b = (E // 4) // CH

    def issue_b(ci, b):
        off = base_b + ci * CH
        pltpu.async_copy(src_hbm.at[_ds8(off, CH)], sbs[b], sems[b])
        pltpu.async_copy(dst_hbm.at[_ds8(off, CH)], dbs[b], sems[b])
        pltpu.async_copy(aeT_hbm.at[_ds8(h * E + off, CH)], abs_[b], sems[b])

    def drain_b(b):
        pltpu.make_async_copy(src_hbm.at[_ds8(0, CH)], sbs[b], sems[b]).wait()
        pltpu.make_async_copy(src_hbm.at[_ds8(0, CH)], dbs[b], sems[b]).wait()
        pltpu.make_async_copy(aeT_hbm.at[_ds8(0, CH)], abs_[b], sems[b]).wait()

    def drain_w(b):
        pltpu.make_async_copy(wbs[b], wT_hbm.at[_ds8(0, CH)],
                              wsems[b]).wait()

    issue_b(0, 0)
    issue_b(1, 1)
    pltpu.sync_copy(asdT_hbm.at[_ds8(h * N, N)], asrc_t)
    pltpu.sync_copy(asdT_hbm.at[_ds8((H + h) * N, N)], adst_t)
    pltpu.sync_copy(mh_hbm.at[_ds8(h * 16, 16)], mh_v)
    _zero_ref(den_acc, N)

    @pl.loop(0, n_b, step=2)
    def _(ci):
        for b in range(2):
            cur = ci + b
            drain_b(b)

            @pl.when(cur >= 2)
            def _():
                drain_w(b)

            @plsc.parallel_loop(0, CH, step=16, unroll=4)
            def _(j):
                sv = sbs[b][pl.ds(j, 16)]
                dv = dbs[b][pl.ds(j, 16)]
                a = (plsc.load_gather(asrc_t, [sv])
                     + plsc.load_gather(adst_t, [dv])
                     + abs_[b][pl.ds(j, 16)])
                a = jnp.maximum(a, a * 0.2)
                wv = jnp.exp(a - mh_v[...])
                wbs[b][pl.ds(j, 16)] = wv
                plsc.addupdate_scatter(den_acc, [dv], wv)

            pltpu.async_copy(wbs[b],
                             wT_hbm.at[_ds8(h * E + base_b + cur * CH, CH)],
                             wsems[b])

            @pl.when(cur + 2 < n_b)
            def _():
                issue_b(cur + 2, b)

    drain_w(0)
    drain_w(1)
    pltpu.sync_copy(den_acc, denP_hbm.at[_ds8(wid * N, N)])


def _sc1(src, dst, eaT, aeT, asdT, mh_tile):
    f = functools.partial(
        pl.kernel,
        out_type=[
            jax.ShapeDtypeStruct((NW * N,), jnp.float32),
            jax.ShapeDtypeStruct((2 * N,), jnp.float32),
            jax.ShapeDtypeStruct((H * E,), jnp.float32),
            jax.ShapeDtypeStruct((NW * N,), jnp.float32),
        ],
        mesh=_mesh(),
        compiler_params=_sc_params(),
        scratch_types=[
            pltpu.VMEM((CH,), jnp.int32),
            pltpu.VMEM((CH,), jnp.int32),
            pltpu.VMEM((CH,), jnp.int32),
            pltpu.VMEM((CH,), jnp.int32),
            pltpu.VMEM((CH,), jnp.float32),
            pltpu.VMEM((CH,), jnp.float32),
            pltpu.VMEM((CH,), jnp.float32),
            pltpu.VMEM((CH,), jnp.float32),
            pltpu.VMEM((CH,), jnp.float32),
            pltpu.VMEM((CH,), jnp.float32),
            pltpu.VMEM((N,), jnp.float32),
            pltpu.VMEM((N,), jnp.float32),
            pltpu.VMEM((N,), jnp.float32),
            pltpu.VMEM((N,), jnp.float32),
            pltpu.VMEM((N,), jnp.float32),
            pltpu.VMEM((16,), jnp.float32),
        ] + [pltpu.SemaphoreType.DMA] * 4,
    )
    return f(_sc1_body)(src, dst, eaT, aeT, asdT, mh_tile)


def _sc2_body(src_hbm, dst_hbm, wT_hbm, xlQ_hbm, outQ_hbm,
              sb0, sb1, db0, db1, wb0, wb1,
              xl0, xl1, xl2, xl3, ac0, ac1, ac2, ac3,
              sem0, sem1):
    wid = lax.axis_index("s") * 2 + lax.axis_index("c")
    h = wid // 4
    xls = (xl0, xl1, xl2, xl3)
    acs = (ac0, ac1, ac2, ac3)
    sbs = (sb0, sb1)
    dbs = (db0, db1)
    wbs = (wb0, wb1)
    sems = (sem0, sem1)
    n_out = E // CH

    def issue(ci, b):
        off = ci * CH
        pltpu.async_copy(src_hbm.at[_ds8(off, CH)], sbs[b], sems[b])
        pltpu.async_copy(dst_hbm.at[_ds8(off, CH)], dbs[b], sems[b])
        pltpu.async_copy(wT_hbm.at[_ds8(h * E + off, CH)], wbs[b], sems[b])

    def drain(b):
        pltpu.make_async_copy(src_hbm.at[_ds8(0, CH)], sbs[b], sems[b]).wait()
        pltpu.make_async_copy(src_hbm.at[_ds8(0, CH)], dbs[b], sems[b]).wait()
        pltpu.make_async_copy(wT_hbm.at[_ds8(0, CH)], wbs[b], sems[b]).wait()

    issue(0, 0)
    issue(1, 1)

    for c in range(QC):
        pltpu.sync_copy(xlQ_hbm.at[_ds8((wid * QC + c) * N, N)], xls[c])
        _zero_ref(acs[c], N)

    @pl.loop(0, n_out, step=2)
    def _(ci):
        for b in range(2):
            cur = ci + b
            drain(b)

            @plsc.parallel_loop(0, CH, step=16, unroll=4)
            def _(j):
                sv = sbs[b][pl.ds(j, 16)]
                dv = dbs[b][pl.ds(j, 16)]
                wv = wbs[b][pl.ds(j, 16)]
                for c in range(QC):
                    g = plsc.load_gather(xls[c], [sv])
                    plsc.addupdate_scatter(acs[c], [dv], g * wv)

            @pl.when(cur + 2 < n_out)
            def _():
                issue(cur + 2, b)

    for c in range(QC):
        pltpu.sync_copy(acs[c], outQ_hbm.at[_ds8((wid * QC + c) * N, N)])


def _sc2(src, dst, wT, xlQ):
    f = functools.partial(
        pl.kernel,
        out_type=jax.ShapeDtypeStruct((NW * QC * N,), jnp.float32),
        mesh=_mesh(),
        compiler_params=_sc_params(),
        scratch_types=[
            pltpu.VMEM((CH,), jnp.int32),
            pltpu.VMEM((CH,), jnp.int32),
            pltpu.VMEM((CH,), jnp.int32),
            pltpu.VMEM((CH,), jnp.int32),
            pltpu.VMEM((CH,), jnp.float32),
            pltpu.VMEM((CH,), jnp.float32),
        ] + [pltpu.VMEM((N,), jnp.float32)] * 8
          + [pltpu.SemaphoreType.DMA] * 2,
    )
    return f(_sc2_body)(src, dst, wT, xlQ)


def _proj_body(x_ref, w_ref, asd_ref, xl_ref, a_ref):
    xb = x_ref[...]
    xl_ref[...] = jnp.dot(xb, w_ref[...], preferred_element_type=jnp.float32)
    a_ref[...] = jnp.dot(xb, asd_ref[...], preferred_element_type=jnp.float32)


def _ae_body(ea_ref, aemat_ref, ae_ref):
    ae_ref[...] = jnp.dot(ea_ref[...], aemat_ref[...],
                          preferred_element_type=jnp.float32)


def _post_body(m_ref, rd_ref, sl_ref, xl_ref, x_ref, b_ref, g_ref, be_ref,
               o_ref):
    out = m_ref[...] * rd_ref[...] + sl_ref[...] * xl_ref[...] + b_ref[...]
    mu = out.mean(-1, keepdims=True)
    var = ((out - mu) ** 2).mean(-1, keepdims=True)
    normed = (out - mu) / jnp.sqrt(var + 1e-5) * g_ref[...] + be_ref[...]
    z = normed + x_ref[...]
    o_ref[...] = jnp.where(z > 0, z, jnp.exp(jnp.minimum(z, 0.0)) - 1.0)


def kernel(x, edge_index, edge_attr, W, W_edge, att_src, att_dst, att_edge,
           bias, gamma, beta):
    n = x.shape[0]
    src = edge_index[0]
    dst = edge_index[1]

    A_s = (W.reshape(D, H, C) * att_src).sum(-1)
    A_d = (W.reshape(D, H, C) * att_dst).sum(-1)
    A_e = (W_edge.reshape(ED, H, C) * att_edge).sum(-1)
    ASD = jnp.concatenate([A_s, A_d], axis=1)

    xl, asd = pl.pallas_call(
        _proj_body,
        grid=(n // BN,),
        in_specs=[
            pl.BlockSpec((BN, D), lambda i: (i, 0)),
            pl.BlockSpec((D, H * C), lambda i: (0, 0)),
            pl.BlockSpec((D, 2 * H), lambda i: (0, 0)),
        ],
        out_specs=[
            pl.BlockSpec((BN, D), lambda i: (i, 0)),
            pl.BlockSpec((BN, 2 * H), lambda i: (i, 0)),
        ],
        out_shape=[
            jax.ShapeDtypeStruct((n, D), jnp.float32),
            jax.ShapeDtypeStruct((n, 2 * H), jnp.float32),
        ],
    )(x, W, ASD)

    ae = pl.pallas_call(
        _ae_body,
        grid=(E // BE,),
        in_specs=[
            pl.BlockSpec((BE, ED), lambda i: (i, 0)),
            pl.BlockSpec((ED, H), lambda i: (0, 0)),
        ],
        out_specs=pl.BlockSpec((BE, H), lambda i: (i, 0)),
        out_shape=jax.ShapeDtypeStruct((E, H), jnp.float32),
    )(edge_attr, A_e)

    asdT = asd.T.reshape(-1)
    aeT = ae.T.reshape(-1)
    eaT = edge_attr.T.reshape(-1)
    xlQ = xl.reshape(n, NW, QC).transpose(1, 2, 0).reshape(-1)

    mh_raw = (jnp.max(asd[:, :H], 0) + jnp.max(asd[:, H:], 0)
              + jnp.max(ae, 0))
    mh = jax.nn.leaky_relu(mh_raw, negative_slope=0.2)
    mh_tile = jnp.tile(mh[:, None], (1, 16)).reshape(-1)

    laP, degP, wT, denP = _sc1(src, dst, eaT, aeT, asdT, mh_tile)
    laP = laP.reshape(NW, n)
    degP = degP.reshape(2, n)
    denP = denP.reshape(NW, n)

    deg = degP.sum(0)
    la = (laP[:16] + laP[16:]) / jnp.clip(deg, 1.0)
    ael = la.T @ A_e
    a_loop = asd[:, :H] + asd[:, H:] + ael
    a_loop = jax.nn.leaky_relu(a_loop, negative_slope=0.2)
    w_loop = jnp.exp(a_loop - mh[None, :])
    denT = denP.reshape(H, 4, n).sum(1) + w_loop.T
    rd2 = 1.0 / denT

    outQ = _sc2(src, dst, wT, xlQ)
    msgP = outQ.reshape(NW, QC, n).transpose(2, 0, 1).reshape(n, D)

    sl = w_loop * rd2.T
    slexp = jnp.repeat(sl, C, axis=1)
    rdexp = jnp.repeat(rd2.T, C, axis=1)

    out = pl.pallas_call(
        _post_body,
        grid=(n // BN,),
        in_specs=[
            pl.BlockSpec((BN, D), lambda i: (i, 0)),
            pl.BlockSpec((BN, D), lambda i: (i, 0)),
            pl.BlockSpec((BN, D), lambda i: (i, 0)),
            pl.BlockSpec((BN, D), lambda i: (i, 0)),
            pl.BlockSpec((BN, D), lambda i: (i, 0)),
            pl.BlockSpec((1, D), lambda i: (0, 0)),
            pl.BlockSpec((1, D), lambda i: (0, 0)),
            pl.BlockSpec((1, D), lambda i: (0, 0)),
        ],
        out_specs=pl.BlockSpec((BN, D), lambda i: (i, 0)),
        out_shape=jax.ShapeDtypeStruct((n, D), jnp.float32),
    )(msgP, rdexp, slexp, xl, x, bias.reshape(1, D), gamma.reshape(1, D),
      beta.reshape(1, D))
    return out


# speedup vs baseline: 2.4281x; 1.3645x over previous
import dataclasses
import functools

import jax
import jax.numpy as jnp
from jax import lax
from jax.experimental import pallas as pl
from jax.experimental.pallas import tpu as pltpu
from jax.experimental.pallas import tpu_sc as plsc

N = 10000
E = 320000
D = 128
H = 8
C = 16
ED = 16

BN = 1000
NP = 10240
BP = 1024
BE = 12800
CH = 4000
NW = 32
QC = 4

def _mesh():
    return plsc.VectorSubcoreMesh(
        core_axis_name="c", subcore_axis_name="s",
        num_cores=2, num_subcores=16)


def _sc_params():
    cp = pltpu.CompilerParams()
    if "needs_layout_passes" in pltpu.CompilerParams.__dataclass_fields__:
        cp = dataclasses.replace(cp, needs_layout_passes=False)
    return cp


def _zero_ref(ref, n):
    @pl.loop(0, n, step=16)
    def _(i):
        ref[pl.ds(i, 16)] = jnp.zeros((16,), jnp.float32)


def _ds8(off, size):
    return pl.ds(pl.multiple_of(off, 8), size)


def _sc1_body(src_hbm, dst_hbm, aeT_hbm, asdT_hbm, mh_hbm,
              aelP_hbm, degP_hbm, wT_hbm, denP_hbm,
              sb0, sb1, db0, db1, ab0, ab1, wb0, wb1,
              ael_acc, deg_acc, asrc_t, adst_t, den_acc, mh_v,
              sem0, sem1, wsem0, wsem1):
    wid = lax.axis_index("s") * 2 + lax.axis_index("c")
    sbs = (sb0, sb1)
    dbs = (db0, db1)
    abs_ = (ab0, ab1)
    wbs = (wb0, wb1)
    sems = (sem0, sem1)
    wsems = (wsem0, wsem1)

    h = wid // 4
    pb = wid % 4
    base_b = pb * (E // 4)
    n_b = (E // 4) // CH

    def issue_b(ci, b):
        off = base_b + ci * CH
        pltpu.async_copy(src_hbm.at[_ds8(off, CH)], sbs[b], sems[b])
        pltpu.async_copy(dst_hbm.at[_ds8(off, CH)], dbs[b], sems[b])
        pltpu.async_copy(aeT_hbm.at[_ds8(h * E + off, CH)], abs_[b], sems[b])

    def drain_b(b):
        pltpu.make_async_copy(src_hbm.at[_ds8(0, CH)], sbs[b], sems[b]).wait()
        pltpu.make_async_copy(src_hbm.at[_ds8(0, CH)], dbs[b], sems[b]).wait()
        pltpu.make_async_copy(aeT_hbm.at[_ds8(0, CH)], abs_[b], sems[b]).wait()

    def drain_w(b):
        pltpu.make_async_copy(wbs[b], wT_hbm.at[_ds8(0, CH)],
                              wsems[b]).wait()

    issue_b(0, 0)
    issue_b(1, 1)
    pltpu.sync_copy(asdT_hbm.at[_ds8(h * NP, N)], asrc_t)
    pltpu.sync_copy(asdT_hbm.at[_ds8((H + h) * NP, N)], adst_t)
    pltpu.sync_copy(mh_hbm.at[_ds8(h * 16, 16)], mh_v)
    _zero_ref(den_acc, N)
    _zero_ref(ael_acc, N)
    _zero_ref(deg_acc, N)

    @pl.loop(0, n_b, step=2)
    def _(ci):
        for b in range(2):
            cur = ci + b
            drain_b(b)

            @pl.when(cur >= 2)
            def _():
                drain_w(b)

            @plsc.parallel_loop(0, CH, step=16, unroll=4)
            def _(j):
                sv = sbs[b][pl.ds(j, 16)]
                dv = dbs[b][pl.ds(j, 16)]
                aev = abs_[b][pl.ds(j, 16)]
                a = (plsc.load_gather(asrc_t, [sv])
                     + plsc.load_gather(adst_t, [dv])
                     + aev)
                a = jnp.maximum(a, a * 0.2)
                wv = jnp.exp(a - mh_v[...])
                wbs[b][pl.ds(j, 16)] = wv
                plsc.addupdate_scatter(den_acc, [dv], wv)
                plsc.addupdate_scatter(ael_acc, [dv], aev)

            @pl.when(h == 0)
            def _():
                @plsc.parallel_loop(0, CH, step=16, unroll=4)
                def _(j):
                    dv = dbs[b][pl.ds(j, 16)]
                    plsc.addupdate_scatter(deg_acc, [dv],
                                           jnp.ones((16,), jnp.float32))

            pltpu.async_copy(wbs[b],
                             wT_hbm.at[_ds8(h * E + base_b + cur * CH, CH)],
                             wsems[b])

            @pl.when(cur + 2 < n_b)
            def _():
                issue_b(cur + 2, b)

    drain_w(0)
    drain_w(1)
    pltpu.sync_copy(den_acc, denP_hbm.at[_ds8(wid * N, N)])
    pltpu.sync_copy(ael_acc, aelP_hbm.at[_ds8(wid * N, N)])

    @pl.when(h == 0)
    def _():
        pltpu.sync_copy(deg_acc, degP_hbm.at[_ds8(pb * N, N)])


def _sc1(src, dst, aeT, asdT, mh_tile):
    f = functools.partial(
        pl.kernel,
        out_type=[
            jax.ShapeDtypeStruct((NW * N,), jnp.float32),
            jax.ShapeDtypeStruct((4 * N,), jnp.float32),
            jax.ShapeDtypeStruct((H * E,), jnp.float32),
            jax.ShapeDtypeStruct((NW * N,), jnp.float32),
        ],
        mesh=_mesh(),
        compiler_params=_sc_params(),
        scratch_types=[
            pltpu.VMEM((CH,), jnp.int32),
            pltpu.VMEM((CH,), jnp.int32),
            pltpu.VMEM((CH,), jnp.int32),
            pltpu.VMEM((CH,), jnp.int32),
            pltpu.VMEM((CH,), jnp.float32),
            pltpu.VMEM((CH,), jnp.float32),
            pltpu.VMEM((CH,), jnp.float32),
            pltpu.VMEM((CH,), jnp.float32),
            pltpu.VMEM((N,), jnp.float32),
            pltpu.VMEM((N,), jnp.float32),
            pltpu.VMEM((N,), jnp.float32),
            pltpu.VMEM((N,), jnp.float32),
            pltpu.VMEM((N,), jnp.float32),
            pltpu.VMEM((16,), jnp.float32),
        ] + [pltpu.SemaphoreType.DMA] * 4,
    )
    return f(_sc1_body)(src, dst, aeT, asdT, mh_tile)


def _sc2_body(src_hbm, dst_hbm, wT_hbm, xlQ_hbm, outQ_hbm,
              sb0, sb1, db0, db1, wb0, wb1,
              xl0, xl1, xl2, xl3, ac0, ac1, ac2, ac3,
              sem0, sem1):
    wid = lax.axis_index("s") * 2 + lax.axis_index("c")
    h = wid // 4
    xls = (xl0, xl1, xl2, xl3)
    acs = (ac0, ac1, ac2, ac3)
    sbs = (sb0, sb1)
    dbs = (db0, db1)
    wbs = (wb0, wb1)
    sems = (sem0, sem1)
    n_out = E // CH

    def issue(ci, b):
        off = ci * CH
        pltpu.async_copy(src_hbm.at[_ds8(off, CH)], sbs[b], sems[b])
        pltpu.async_copy(dst_hbm.at[_ds8(off, CH)], dbs[b], sems[b])
        pltpu.async_copy(wT_hbm.at[_ds8(h * E + off, CH)], wbs[b], sems[b])

    def drain(b):
        pltpu.make_async_copy(src_hbm.at[_ds8(0, CH)], sbs[b], sems[b]).wait()
        pltpu.make_async_copy(src_hbm.at[_ds8(0, CH)], dbs[b], sems[b]).wait()
        pltpu.make_async_copy(wT_hbm.at[_ds8(0, CH)], wbs[b], sems[b]).wait()

    issue(0, 0)
    issue(1, 1)

    for c in range(QC):
        pltpu.sync_copy(xlQ_hbm.at[_ds8((wid * QC + c) * NP, N)], xls[c])
        _zero_ref(acs[c], N)

    @pl.loop(0, n_out, step=2)
    def _(ci):
        for b in range(2):
            cur = ci + b
            drain(b)

            @plsc.parallel_loop(0, CH, step=16, unroll=4)
            def _(j):
                sv = sbs[b][pl.ds(j, 16)]
                dv = dbs[b][pl.ds(j, 16)]
                wv = wbs[b][pl.ds(j, 16)]
                for c in range(QC):
                    g = plsc.load_gather(xls[c], [sv])
                    plsc.addupdate_scatter(acs[c], [dv], g * wv)

            @pl.when(cur + 2 < n_out)
            def _():
                issue(cur + 2, b)

    for c in range(QC):
        pltpu.sync_copy(acs[c], outQ_hbm.at[_ds8((wid * QC + c) * N, N)])


def _sc2(src, dst, wT, xlQ):
    f = functools.partial(
        pl.kernel,
        out_type=jax.ShapeDtypeStruct((NW * QC * N,), jnp.float32),
        mesh=_mesh(),
        compiler_params=_sc_params(),
        scratch_types=[
            pltpu.VMEM((CH,), jnp.int32),
            pltpu.VMEM((CH,), jnp.int32),
            pltpu.VMEM((CH,), jnp.int32),
            pltpu.VMEM((CH,), jnp.int32),
            pltpu.VMEM((CH,), jnp.float32),
            pltpu.VMEM((CH,), jnp.float32),
        ] + [pltpu.VMEM((N,), jnp.float32)] * 8
          + [pltpu.SemaphoreType.DMA] * 2,
    )
    return f(_sc2_body)(src, dst, wT, xlQ)


_DN_T = (((0,), (1,)), ((), ()))


def _proj_body(x_ref, w_ref, asd_ref, xl_ref, xlt_ref, at_ref):
    i = pl.program_id(0)
    xb = x_ref[...]
    xl_ref[...] = jnp.dot(xb, w_ref[...], preferred_element_type=jnp.float32)
    xlt_ref[:, pl.ds(pl.multiple_of(i * BP, 128), BP)] = lax.dot_general(
        w_ref[...], xb, _DN_T, preferred_element_type=jnp.float32)
    at_ref[:, pl.ds(pl.multiple_of(i * BP, 128), BP)] = lax.dot_general(
        asd_ref[...], xb, _DN_T, preferred_element_type=jnp.float32)


def _ae_body(ea_ref, aemat_ref, aet_ref):
    i = pl.program_id(0)
    aet_ref[:, pl.ds(pl.multiple_of(i * BE, 128), BE)] = lax.dot_general(
        aemat_ref[...], ea_ref[...], _DN_T,
        preferred_element_type=jnp.float32)


def _post_body(m_ref, rd_ref, sl_ref, xl_ref, x_ref, b_ref, g_ref, be_ref,
               o_ref):
    out = m_ref[...] * rd_ref[...] + sl_ref[...] * xl_ref[...] + b_ref[...]
    mu = out.mean(-1, keepdims=True)
    var = ((out - mu) ** 2).mean(-1, keepdims=True)
    normed = (out - mu) / jnp.sqrt(var + 1e-5) * g_ref[...] + be_ref[...]
    z = normed + x_ref[...]
    o_ref[...] = jnp.where(z > 0, z, jnp.exp(jnp.minimum(z, 0.0)) - 1.0)


def kernel(x, edge_index, edge_attr, W, W_edge, att_src, att_dst, att_edge,
           bias, gamma, beta):
    n = x.shape[0]
    src = edge_index[0]
    dst = edge_index[1]

    A_s = (W.reshape(D, H, C) * att_src).sum(-1)
    A_d = (W.reshape(D, H, C) * att_dst).sum(-1)
    A_e = (W_edge.reshape(ED, H, C) * att_edge).sum(-1)
    ASD = jnp.concatenate([A_s, A_d], axis=1)

    xp = jnp.pad(x, ((0, NP - n), (0, 0)))
    xl, xlT, asdT2 = pl.pallas_call(
        _proj_body,
        grid=(NP // BP,),
        in_specs=[
            pl.BlockSpec((BP, D), lambda i: (i, 0)),
            pl.BlockSpec((D, H * C), lambda i: (0, 0)),
            pl.BlockSpec((D, 2 * H), lambda i: (0, 0)),
        ],
        out_specs=[
            pl.BlockSpec((BP, D), lambda i: (i, 0)),
            pl.BlockSpec((D, NP), lambda i: (0, 0)),
            pl.BlockSpec((2 * H, NP), lambda i: (0, 0)),
        ],
        out_shape=[
            jax.ShapeDtypeStruct((NP, D), jnp.float32),
            jax.ShapeDtypeStruct((D, NP), jnp.float32),
            jax.ShapeDtypeStruct((2 * H, NP), jnp.float32),
        ],
    )(xp, W, ASD)

    aeT2 = pl.pallas_call(
        _ae_body,
        grid=(E // BE,),
        in_specs=[
            pl.BlockSpec((BE, ED), lambda i: (i, 0)),
            pl.BlockSpec((ED, H), lambda i: (0, 0)),
        ],
        out_specs=pl.BlockSpec((H, E), lambda i: (0, 0)),
        out_shape=jax.ShapeDtypeStruct((H, E), jnp.float32),
    )(edge_attr, A_e)

    asdT = asdT2.reshape(-1)
    aeT = aeT2.reshape(-1)
    xlQ = xlT.reshape(-1)

    mh_raw = (jnp.max(asdT2[:H], 1) + jnp.max(asdT2[H:], 1)
              + jnp.max(aeT2, 1))
    mh = jax.nn.leaky_relu(mh_raw, negative_slope=0.2)
    mh_tile = jnp.tile(mh[:, None], (1, 16)).reshape(-1)

    aelP, degP, wT, denP = _sc1(src, dst, aeT, asdT, mh_tile)

    deg = degP.reshape(4, n).sum(0)
    aelT = (aelP.reshape(H, 4, n).sum(1)
            / jnp.clip(deg, 1.0)[None, :])
    a_loopT = asdT2[:H, :n] + asdT2[H:, :n] + aelT
    a_loopT = jax.nn.leaky_relu(a_loopT, negative_slope=0.2)
    w_loopT = jnp.exp(a_loopT - mh[:, None])
    denT = denP.reshape(H, 4, n).sum(1) + w_loopT
    rd2 = 1.0 / denT

    outQ = _sc2(src, dst, wT, xlQ)
    msgP = outQ.reshape(D, n).T

    slT = w_loopT * rd2
    slexp = jnp.repeat(slT.T, C, axis=1)
    rdexp = jnp.repeat(rd2.T, C, axis=1)

    out = pl.pallas_call(
        _post_body,
        grid=(n // BN,),
        in_specs=[
            pl.BlockSpec((BN, D), lambda i: (i, 0)),
            pl.BlockSpec((BN, D), lambda i: (i, 0)),
            pl.BlockSpec((BN, D), lambda i: (i, 0)),
            pl.BlockSpec((BN, D), lambda i: (i, 0)),
            pl.BlockSpec((BN, D), lambda i: (i, 0)),
            pl.BlockSpec((1, D), lambda i: (0, 0)),
            pl.BlockSpec((1, D), lambda i: (0, 0)),
            pl.BlockSpec((1, D), lambda i: (0, 0)),
        ],
        out_specs=pl.BlockSpec((BN, D), lambda i: (i, 0)),
        out_shape=jax.ShapeDtypeStruct((n, D), jnp.float32),
    )(msgP, rdexp, slexp, xl, x, bias.reshape(1, D), gamma.reshape(1, D),
      beta.reshape(1, D))
    return out
